# Initial kernel scaffold; baseline (speedup 1.0000x reference)
import jax
import jax.numpy as jnp
from jax.experimental import pallas as pl


def kernel(patient_id, patient_age, patient_dialysis_freq, patient_dialysis_latitude, patient_dialysis_longitude, patient_skills, emb_table):
    raise NotImplementedError("write your pallas kernel here")


# speedup vs baseline: 5.1863x; 5.1863x over previous
import functools

import jax
import jax.numpy as jnp
from jax import lax
from jax.experimental import pallas as pl
from jax.experimental.pallas import tpu as pltpu
from jax.experimental.pallas import tpu_sc as plsc

B = 16384
D = 32
NSK = 8
NUM_SKILL_COLS = 65
OUT_D = D + 4 + NUM_SKILL_COLS
NW = 32
BPW = B // NW
AGE_MEAN = 45.0
AGE_STD = 20.0


def _body(table_hbm, pid_hbm, age_hbm, freq_hbm, lat_hbm, lon_hbm,
          skills_hbm, out_hbm, idx_v, rows_v, skills_v, age_v, freq_v,
          lat_v, lon_v, slab, sem):
    c = lax.axis_index("c")
    s = lax.axis_index("s")
    wid = s * 2 + c
    base = wid * BPW

    pltpu.sync_copy(pid_hbm.at[pl.ds(base, BPW)], idx_v)
    gather = pltpu.async_copy(table_hbm.at[idx_v], rows_v, sem)
    pltpu.sync_copy(skills_hbm.at[pl.ds(base * NSK, BPW * NSK)], skills_v)
    pltpu.sync_copy(age_hbm.at[pl.ds(base, BPW)], age_v)
    pltpu.sync_copy(freq_hbm.at[pl.ds(base, BPW)], freq_v)
    pltpu.sync_copy(lat_hbm.at[pl.ds(base, BPW)], lat_v)
    pltpu.sync_copy(lon_hbm.at[pl.ds(base, BPW)], lon_v)

    iota = lax.iota(jnp.int32, 16)
    zeros = jnp.zeros((16,), jnp.float32)
    ones = jnp.ones((16,), jnp.float32)

    def zero_body(i, carry):
        for j in range(4):
            r = (i * 4 + j) * OUT_D
            slab[pl.ds(r + 36, 16)] = zeros
            slab[pl.ds(r + 52, 16)] = zeros
            slab[pl.ds(r + 68, 16)] = zeros
            slab[pl.ds(r + 84, 16)] = zeros
            slab[pl.ds(r + 85, 16)] = zeros
        return carry

    lax.fori_loop(0, BPW // 4, zero_body, 0)

    rowoff = lax.shift_right_logical(iota, 3) * OUT_D + 36

    def mh_body(i, carry):
        sk = skills_v[pl.ds(i * 16, 16)]
        flat = i * (2 * OUT_D) + rowoff + sk
        plsc.store_scatter(slab, [flat], ones)
        return carry

    lax.fori_loop(0, (BPW * NSK) // 16, mh_body, 0)

    iota_row = iota * OUT_D

    def scal_body(i, carry):
        b16 = i * 16
        flat = b16 * OUT_D + iota_row + D
        a = (age_v[pl.ds(b16, 16)] - AGE_MEAN) / AGE_STD
        plsc.store_scatter(slab, [flat], a)
        plsc.store_scatter(slab, [flat + 1], freq_v[pl.ds(b16, 16)])
        plsc.store_scatter(slab, [flat + 2], lat_v[pl.ds(b16, 16)])
        plsc.store_scatter(slab, [flat + 3], lon_v[pl.ds(b16, 16)])
        return carry

    lax.fori_loop(0, BPW // 16, scal_body, 0)

    gather.wait()

    def emb_body(i, carry):
        for j in range(2):
            r = i * 2 + j
            o = r * OUT_D
            slab[pl.ds(o, 16)] = rows_v[r, pl.ds(0, 16)]
            slab[pl.ds(o + 16, 16)] = rows_v[r, pl.ds(16, 16)]
        return carry

    lax.fori_loop(0, BPW // 2, emb_body, 0)

    pltpu.sync_copy(slab, out_hbm.at[pl.ds(base * OUT_D, BPW * OUT_D)])


_patient_sc = functools.partial(
    pl.kernel,
    out_type=jax.ShapeDtypeStruct((B * OUT_D,), jnp.float32),
    mesh=plsc.VectorSubcoreMesh(core_axis_name="c", subcore_axis_name="s"),
    compiler_params=pltpu.CompilerParams(
        needs_layout_passes=False, use_tc_tiling_on_sc=False),
    scratch_types=[
        pltpu.VMEM((BPW,), jnp.int32),
        pltpu.VMEM((BPW, D), jnp.float32),
        pltpu.VMEM((BPW * NSK,), jnp.int32),
        pltpu.VMEM((BPW,), jnp.float32),
        pltpu.VMEM((BPW,), jnp.float32),
        pltpu.VMEM((BPW,), jnp.float32),
        pltpu.VMEM((BPW,), jnp.float32),
        pltpu.VMEM((BPW * OUT_D,), jnp.float32),
        pltpu.SemaphoreType.DMA,
    ],
)(_body)


@jax.jit
def kernel(patient_id, patient_age, patient_dialysis_freq,
           patient_dialysis_latitude, patient_dialysis_longitude,
           patient_skills, emb_table):
    pid = patient_id.astype(jnp.int32)
    skills_flat = patient_skills.astype(jnp.int32).reshape(-1)
    flat = _patient_sc(emb_table, pid, patient_age, patient_dialysis_freq,
                       patient_dialysis_latitude,
                       patient_dialysis_longitude, skills_flat)
    return flat.reshape(B, OUT_D)


# speedup vs baseline: 5.4419x; 1.0493x over previous
import functools

import jax
import jax.numpy as jnp
from jax import lax
from jax.experimental import pallas as pl
from jax.experimental.pallas import tpu as pltpu
from jax.experimental.pallas import tpu_sc as plsc

B = 16384
D = 32
NSK = 8
NUM_SKILL_COLS = 65
OUT_D = D + 4 + NUM_SKILL_COLS
NW = 32
BPW = B // NW
AGE_MEAN = 45.0
AGE_STD = 20.0


def _body(table_hbm, pid_hbm, age_hbm, freq_hbm, lat_hbm, lon_hbm,
          skills_hbm, out_hbm, idx_v, rows_v, skills_v, age_v, freq_v,
          lat_v, lon_v, slab, sem):
    c = lax.axis_index("c")
    s = lax.axis_index("s")
    wid = s * 2 + c
    base = wid * BPW

    pltpu.sync_copy(pid_hbm.at[pl.ds(base, BPW)], idx_v)
    gather = pltpu.async_copy(table_hbm.at[idx_v], rows_v, sem)
    pltpu.sync_copy(skills_hbm.at[pl.ds(base * NSK, BPW * NSK)], skills_v)
    pltpu.sync_copy(age_hbm.at[pl.ds(base, BPW)], age_v)
    pltpu.sync_copy(freq_hbm.at[pl.ds(base, BPW)], freq_v)
    pltpu.sync_copy(lat_hbm.at[pl.ds(base, BPW)], lat_v)
    pltpu.sync_copy(lon_hbm.at[pl.ds(base, BPW)], lon_v)

    iota = lax.iota(jnp.int32, 16)
    zeros = jnp.zeros((16,), jnp.float32)
    ones = jnp.ones((16,), jnp.float32)

    @plsc.parallel_loop(0, BPW, unroll=8)
    def _zero(r):
        o = r * OUT_D
        slab[pl.ds(o + 36, 16)] = zeros
        slab[pl.ds(o + 52, 16)] = zeros
        slab[pl.ds(o + 68, 16)] = zeros
        slab[pl.ds(o + 84, 16)] = zeros
        slab[pl.ds(o + 85, 16)] = zeros

    rowoff = lax.shift_right_logical(iota, 3) * OUT_D + 36

    @plsc.parallel_loop(0, (BPW * NSK) // 16, unroll=8)
    def _mh(i):
        sk = skills_v[pl.ds(i * 16, 16)]
        flat = i * (2 * OUT_D) + rowoff + sk
        plsc.store_scatter(slab, [flat], ones)

    iota_row = iota * OUT_D

    @plsc.parallel_loop(0, BPW // 16, unroll=4)
    def _scal(i):
        b16 = i * 16
        flat = b16 * OUT_D + iota_row + D
        a = (age_v[pl.ds(b16, 16)] - AGE_MEAN) / AGE_STD
        plsc.store_scatter(slab, [flat], a)
        plsc.store_scatter(slab, [flat + 1], freq_v[pl.ds(b16, 16)])
        plsc.store_scatter(slab, [flat + 2], lat_v[pl.ds(b16, 16)])
        plsc.store_scatter(slab, [flat + 3], lon_v[pl.ds(b16, 16)])

    gather.wait()

    @plsc.parallel_loop(0, BPW, unroll=8)
    def _emb(r):
        o = r * OUT_D
        slab[pl.ds(o, 16)] = rows_v[r, pl.ds(0, 16)]
        slab[pl.ds(o + 16, 16)] = rows_v[r, pl.ds(16, 16)]

    pltpu.sync_copy(slab, out_hbm.at[pl.ds(base * OUT_D, BPW * OUT_D)])


_patient_sc = functools.partial(
    pl.kernel,
    out_type=jax.ShapeDtypeStruct((B * OUT_D,), jnp.float32),
    mesh=plsc.VectorSubcoreMesh(core_axis_name="c", subcore_axis_name="s"),
    compiler_params=pltpu.CompilerParams(
        needs_layout_passes=False, use_tc_tiling_on_sc=False),
    scratch_types=[
        pltpu.VMEM((BPW,), jnp.int32),
        pltpu.VMEM((BPW, D), jnp.float32),
        pltpu.VMEM((BPW * NSK,), jnp.int32),
        pltpu.VMEM((BPW,), jnp.float32),
        pltpu.VMEM((BPW,), jnp.float32),
        pltpu.VMEM((BPW,), jnp.float32),
        pltpu.VMEM((BPW,), jnp.float32),
        pltpu.VMEM((BPW * OUT_D,), jnp.float32),
        pltpu.SemaphoreType.DMA,
    ],
)(_body)


@jax.jit
def kernel(patient_id, patient_age, patient_dialysis_freq,
           patient_dialysis_latitude, patient_dialysis_longitude,
           patient_skills, emb_table):
    pid = patient_id.astype(jnp.int32)
    skills_flat = patient_skills.astype(jnp.int32).reshape(-1)
    flat = _patient_sc(emb_table, pid, patient_age, patient_dialysis_freq,
                       patient_dialysis_latitude,
                       patient_dialysis_longitude, skills_flat)
    return flat.reshape(B, OUT_D)
